# baseline (speedup 1.0000x reference)
import functools

import jax
import jax.numpy as jnp
from jax import lax
from jax.experimental import pallas as pl
from jax.experimental.pallas import tpu as pltpu
from jax.experimental.pallas import tpu_sc as plsc

_N = 10000
_E = 320000
_D = 128
_H = 128
_NC = 2
_NS = 16
_NW = _NC * _NS
_CH = 128
_CPT = 80
_NCHUNK = _CPT * _NW
_EPAD = _NCHUNK * _CH
_FILL = 10080
_DR = 80
_DBASE = _N
_NA = 10240
_RPT = _NA // _NS
_NP = _DR * _D


def _sc_scatter(x, rows, cols, zrows):
    mesh = plsc.VectorSubcoreMesh(core_axis_name="c", subcore_axis_name="s")

    @functools.partial(
        pl.kernel,
        out_type=pltpu.HBM((_NC, _NA, _D), jnp.float32),
        mesh=mesh,
        compiler_params=pltpu.CompilerParams(needs_layout_passes=False),
        scratch_types=[
            pltpu.VMEM((_CH,), jnp.int32),
            pltpu.VMEM((_CH,), jnp.int32),
            pltpu.VMEM((_CH, _D), jnp.float32),
            pltpu.VMEM((_CH,), jnp.int32),
            pltpu.VMEM((_CH,), jnp.int32),
            pltpu.VMEM((_CH, _D), jnp.float32),
            pltpu.VMEM((_DR, _D), jnp.float32),
            pltpu.VMEM((_DR,), jnp.int32),
            pltpu.VMEM_SHARED((_NA, _D), jnp.float32),
            pltpu.SemaphoreType.DMA,
            pltpu.SemaphoreType.DMA,
            pltpu.SemaphoreType.DMA,
            pltpu.SemaphoreType.DMA,
            pltpu.SemaphoreType.DMA,
            pltpu.SemaphoreType.DMA,
        ],
    )
    def k(x_hbm, rows_hbm, cols_hbm, z_hbm, out_hbm,
          rv0, cv0, g0, rv1, cv1, g1, deg_v, didx,
          acc_sh, sg0, sg1, ss0, ss1, si0, si1):
        cid = lax.axis_index("c")
        sid = lax.axis_index("s")
        wid = cid * _NS + sid
        pltpu.sync_copy(z_hbm, acc_sh.at[pl.ds(sid * _RPT, _RPT)])
        pltpu.sync_copy(z_hbm.at[pl.ds(0, _DR)], deg_v)
        iota16 = lax.iota(jnp.int32, 16)
        for m in range(_DR // 16):
            didx[pl.ds(m * 16, 16)] = _DBASE + m * 16 + iota16
        plsc.subcore_barrier()

        cb = wid * _CPT
        masks = [iota16 == kk for kk in range(16)]
        ones16 = jnp.full((16,), 1.0, jnp.float32)

        def count_degrees(idx_ref):
            for kk in range(_CH // 16):
                r16 = idx_ref[pl.ds(kk * 16, 16)]
                hgh = lax.shift_right_logical(r16, 7)
                hgl = lax.bitwise_and(r16, 127)
                for mm in masks:
                    plsc.addupdate_scatter(deg_v, [hgh, hgl], ones16,
                                           mask=mm)

        def idx_start(j, rv, cv, si):
            off = pl.multiple_of((cb + j) * _CH, 8)
            pltpu.make_async_copy(rows_hbm.at[pl.ds(off, _CH)], rv, si).start()
            pltpu.make_async_copy(cols_hbm.at[pl.ds(off, _CH)], cv, si).start()

        def idx_wait(j, rv, cv, si):
            off = pl.multiple_of((cb + j) * _CH, 8)
            pltpu.make_async_copy(rows_hbm.at[pl.ds(off, _CH)], rv, si).wait()
            pltpu.make_async_copy(cols_hbm.at[pl.ds(off, _CH)], cv, si).wait()

        def gather_start(cv, g, sg):
            pltpu.make_async_copy(x_hbm.at[cv], g, sg).start()

        def gather_wait(cv, g, sg):
            pltpu.make_async_copy(x_hbm.at[cv], g, sg).wait()

        def scat_start(g, rv, ss):
            pltpu.make_async_copy(g, acc_sh.at[rv], ss).start(add=True)

        def scat_wait(g, rv, ss):
            pltpu.make_async_copy(g, acc_sh.at[rv], ss).wait()

        idx_start(0, rv0, cv0, si0)
        idx_wait(0, rv0, cv0, si0)
        gather_start(cv0, g0, sg0)
        idx_start(1, rv1, cv1, si1)
        count_degrees(rv0)

        def step(t, carry):
            j0 = 2 * t
            idx_wait(j0 + 1, rv1, cv1, si1)
            gather_start(cv1, g1, sg1)
            count_degrees(rv1)
            gather_wait(cv0, g0, sg0)
            scat_start(g0, rv0, ss0)
            scat_wait(g0, rv0, ss0)
            idx_start(j0 + 2, rv0, cv0, si0)
            idx_wait(j0 + 2, rv0, cv0, si0)
            gather_start(cv0, g0, sg0)
            count_degrees(rv0)
            gather_wait(cv1, g1, sg1)
            scat_start(g1, rv1, ss1)
            scat_wait(g1, rv1, ss1)
            idx_start(j0 + 3, rv1, cv1, si1)
            return carry

        lax.fori_loop(0, _CPT // 2 - 1, step, 0)
        idx_wait(_CPT - 1, rv1, cv1, si1)
        gather_start(cv1, g1, sg1)
        count_degrees(rv1)
        gather_wait(cv0, g0, sg0)
        scat_start(g0, rv0, ss0)
        scat_wait(g0, rv0, ss0)
        gather_wait(cv1, g1, sg1)
        scat_start(g1, rv1, ss1)
        scat_wait(g1, rv1, ss1)

        pltpu.sync_copy(deg_v, acc_sh.at[didx], add=True)
        plsc.subcore_barrier()
        pltpu.sync_copy(acc_sh.at[pl.ds(sid * _RPT, _RPT)],
                        out_hbm.at[cid, pl.ds(sid * _RPT, _RPT)])

    return k(x, rows, cols, zrows)


_BLK = 1024


def _tc_dense(acc, deg, x, W, b2):
    def body(acc_ref, deg_ref, x_ref, w_ref, b_ref, o_ref):
        d = jnp.maximum(deg_ref[...], 1.0)
        neigh = (acc_ref[0] + acc_ref[1]) / d
        h = (jnp.dot(x_ref[...], w_ref[:_D, :],
                     preferred_element_type=jnp.float32,
                     precision=lax.Precision.HIGHEST)
             + jnp.dot(neigh, w_ref[_D:, :],
                       preferred_element_type=jnp.float32,
                       precision=lax.Precision.HIGHEST)
             + b_ref[...])
        o_ref[...] = jnp.maximum(h, 0.0)

    return pl.pallas_call(
        body,
        grid=(_N // _BLK + 1,),
        in_specs=[
            pl.BlockSpec((_NC, _BLK, _D), lambda i: (0, i, 0)),
            pl.BlockSpec((_BLK, 1), lambda i: (i, 0)),
            pl.BlockSpec((_BLK, _D), lambda i: (i, 0)),
            pl.BlockSpec((2 * _D, _H), lambda i: (0, 0)),
            pl.BlockSpec((1, _H), lambda i: (0, 0)),
        ],
        out_specs=pl.BlockSpec((_BLK, _H), lambda i: (i, 0)),
        out_shape=jax.ShapeDtypeStruct((_N, _H), jnp.float32),
    )(acc, deg, x, W, b2)


def kernel(x, rows, cols, W, b):
    fill = _EPAD - _E
    frows = _FILL + jnp.arange(fill, dtype=jnp.int32) % 160
    rows_p = jnp.concatenate([rows, frows])
    cols_p = jnp.concatenate([cols, jnp.zeros((fill,), jnp.int32)])
    zrows = jnp.zeros((_RPT, _D), jnp.float32)
    acc = _sc_scatter(x, rows_p, cols_p, zrows)
    dd = acc[0, _DBASE:_DBASE + _DR, :] + acc[1, _DBASE:_DBASE + _DR, :]
    deg = dd.reshape(_NP, 1)
    return _tc_dense(acc, deg, x, W, b.reshape(1, _H))


# speedup vs baseline: 3.3967x; 3.3967x over previous
import functools

import jax
import jax.numpy as jnp
from jax import lax
from jax.experimental import pallas as pl
from jax.experimental.pallas import tpu as pltpu
from jax.experimental.pallas import tpu_sc as plsc

_N = 10000
_E = 320000
_D = 128
_H = 128
_NC = 2
_NS = 16
_NW = _NC * _NS
_CH = 128
_CPT = 80
_NCHUNK = _CPT * _NW
_EPAD = _NCHUNK * _CH
_FILL = 10080
_DR = 80
_DBASE = _N
_NA = 10240
_RPT = _NA // _NS
_NP = _DR * _D


def _sc_scatter(x, rows, cols, zrows):
    mesh = plsc.VectorSubcoreMesh(core_axis_name="c", subcore_axis_name="s")

    @functools.partial(
        pl.kernel,
        out_type=pltpu.HBM((_NC, _NA, _D), jnp.float32),
        mesh=mesh,
        compiler_params=pltpu.CompilerParams(needs_layout_passes=False),
        scratch_types=[
            pltpu.VMEM((_CH,), jnp.int32),
            pltpu.VMEM((_CH,), jnp.int32),
            pltpu.VMEM((_CH, _D), jnp.float32),
            pltpu.VMEM((_CH,), jnp.int32),
            pltpu.VMEM((_CH,), jnp.int32),
            pltpu.VMEM((_CH, _D), jnp.float32),
            pltpu.VMEM((_DR, _D), jnp.float32),
            pltpu.VMEM((_DR,), jnp.int32),
            pltpu.VMEM_SHARED((_NA, _D), jnp.float32),
            pltpu.SemaphoreType.DMA,
            pltpu.SemaphoreType.DMA,
            pltpu.SemaphoreType.DMA,
            pltpu.SemaphoreType.DMA,
            pltpu.SemaphoreType.DMA,
            pltpu.SemaphoreType.DMA,
        ],
    )
    def k(x_hbm, rows_hbm, cols_hbm, z_hbm, out_hbm,
          rv0, cv0, g0, rv1, cv1, g1, deg_v, didx,
          acc_sh, sg0, sg1, ss0, ss1, si0, si1):
        cid = lax.axis_index("c")
        sid = lax.axis_index("s")
        wid = cid * _NS + sid
        pltpu.sync_copy(z_hbm, acc_sh.at[pl.ds(sid * _RPT, _RPT)])
        pltpu.sync_copy(z_hbm.at[pl.ds(0, _DR)], deg_v)
        iota16 = lax.iota(jnp.int32, 16)
        for m in range(_DR // 16):
            didx[pl.ds(m * 16, 16)] = _DBASE + m * 16 + iota16
        plsc.subcore_barrier()

        cb = wid * _CPT
        masks = [iota16 == kk for kk in range(16)]
        ones16 = jnp.full((16,), 1.0, jnp.float32)

        def count_degrees(idx_ref):
            for kk in range(_CH // 16):
                r16 = idx_ref[pl.ds(kk * 16, 16)]
                hgh = lax.shift_right_logical(r16, 7)
                hgl = lax.bitwise_and(r16, 127)
                for mm in masks:
                    plsc.addupdate_scatter(deg_v, [hgh, hgl], ones16,
                                           mask=mm)

        def idx_start(j, rv, cv, si):
            off = pl.multiple_of((cb + j) * _CH, 8)
            pltpu.make_async_copy(rows_hbm.at[pl.ds(off, _CH)], rv, si).start()
            pltpu.make_async_copy(cols_hbm.at[pl.ds(off, _CH)], cv, si).start()

        def idx_wait(j, rv, cv, si):
            off = pl.multiple_of((cb + j) * _CH, 8)
            pltpu.make_async_copy(rows_hbm.at[pl.ds(off, _CH)], rv, si).wait()
            pltpu.make_async_copy(cols_hbm.at[pl.ds(off, _CH)], cv, si).wait()

        def gather_start(cv, g, sg):
            pltpu.make_async_copy(x_hbm.at[cv], g, sg).start()

        def gather_wait(cv, g, sg):
            pltpu.make_async_copy(x_hbm.at[cv], g, sg).wait()

        def scat_start(g, rv, ss):
            pltpu.make_async_copy(g, acc_sh.at[rv], ss).start(add=True)

        def scat_wait(g, rv, ss):
            pltpu.make_async_copy(g, acc_sh.at[rv], ss).wait()

        idx_start(0, rv0, cv0, si0)
        idx_wait(0, rv0, cv0, si0)
        gather_start(cv0, g0, sg0)
        idx_start(1, rv1, cv1, si1)
        count_degrees(rv0)

        def step(t, carry):
            j0 = 2 * t
            idx_wait(j0 + 1, rv1, cv1, si1)
            gather_start(cv1, g1, sg1)
            count_degrees(rv1)
            gather_wait(cv0, g0, sg0)
            scat_start(g0, rv0, ss0)
            scat_wait(g0, rv0, ss0)
            idx_start(j0 + 2, rv0, cv0, si0)
            idx_wait(j0 + 2, rv0, cv0, si0)
            gather_start(cv0, g0, sg0)
            count_degrees(rv0)
            gather_wait(cv1, g1, sg1)
            scat_start(g1, rv1, ss1)
            scat_wait(g1, rv1, ss1)
            idx_start(j0 + 3, rv1, cv1, si1)
            return carry

        lax.fori_loop(0, _CPT // 2 - 1, step, 0)
        idx_wait(_CPT - 1, rv1, cv1, si1)
        gather_start(cv1, g1, sg1)
        count_degrees(rv1)
        gather_wait(cv0, g0, sg0)
        scat_start(g0, rv0, ss0)
        scat_wait(g0, rv0, ss0)
        gather_wait(cv1, g1, sg1)
        scat_start(g1, rv1, ss1)
        scat_wait(g1, rv1, ss1)

        pltpu.sync_copy(deg_v, acc_sh.at[didx], add=True)
        plsc.subcore_barrier()
        pltpu.sync_copy(acc_sh.at[pl.ds(sid * _RPT, _RPT)],
                        out_hbm.at[cid, pl.ds(sid * _RPT, _RPT)])

    return k(x, rows, cols, zrows)


_BLK = 1024


def _tc_dense(acc, deg, x, W, b2):
    def body(acc_ref, deg_ref, x_ref, w_ref, b_ref, o_ref):
        d = jnp.maximum(deg_ref[...], 1.0)
        neigh = (acc_ref[0] + acc_ref[1]) / d
        h = (jnp.dot(x_ref[...], w_ref[:_D, :],
                     preferred_element_type=jnp.float32,
                     precision=lax.Precision.HIGHEST)
             + jnp.dot(neigh, w_ref[_D:, :],
                       preferred_element_type=jnp.float32,
                       precision=lax.Precision.HIGHEST)
             + b_ref[...])
        o_ref[...] = jnp.maximum(h, 0.0)

    return pl.pallas_call(
        body,
        grid=(_N // _BLK + 1,),
        in_specs=[
            pl.BlockSpec((_NC, _BLK, _D), lambda i: (0, i, 0)),
            pl.BlockSpec((_BLK, 1), lambda i: (i, 0)),
            pl.BlockSpec((_BLK, _D), lambda i: (i, 0)),
            pl.BlockSpec((2 * _D, _H), lambda i: (0, 0)),
            pl.BlockSpec((1, _H), lambda i: (0, 0)),
        ],
        out_specs=pl.BlockSpec((_BLK, _H), lambda i: (i, 0)),
        out_shape=jax.ShapeDtypeStruct((_N, _H), jnp.float32),
    )(acc, deg, x, W, b2)


def kernel(x, rows, cols, W, b):
    fill = _EPAD - _E
    frows = _FILL + jnp.arange(fill, dtype=jnp.int32) % 160
    rows_p = jnp.concatenate([rows, frows])
    cols_p = jnp.concatenate([cols, jnp.arange(fill, dtype=jnp.int32) % _N])
    zrows = jnp.zeros((_RPT, _D), jnp.float32)
    acc = _sc_scatter(x, rows_p, cols_p, zrows)
    dd = acc[0, _DBASE:_DBASE + _DR, :] + acc[1, _DBASE:_DBASE + _DR, :]
    deg = dd.reshape(_NP, 1)
    return _tc_dense(acc, deg, x, W, b.reshape(1, _H))


# speedup vs baseline: 3.4097x; 1.0038x over previous
import functools

import jax
import jax.numpy as jnp
from jax import lax
from jax.experimental import pallas as pl
from jax.experimental.pallas import tpu as pltpu
from jax.experimental.pallas import tpu_sc as plsc

_N = 10000
_E = 320000
_D = 128
_H = 128
_NC = 2
_NS = 16
_NW = _NC * _NS
_CH = 128
_NCHUNK = _E // _CH
_CPT = _NCHUNK // _NW
_XC = _NCHUNK - _CPT * _NW
_DR = 80
_DBASE = _N
_NA = 10240
_RPT = _NA // _NS
_NP = _DR * _D


def _sc_scatter(x, rows, cols, zrows):
    mesh = plsc.VectorSubcoreMesh(core_axis_name="c", subcore_axis_name="s")

    @functools.partial(
        pl.kernel,
        out_type=pltpu.HBM((_NC, _NA, _D), jnp.float32),
        mesh=mesh,
        compiler_params=pltpu.CompilerParams(needs_layout_passes=False),
        scratch_types=[
            pltpu.VMEM((_CH,), jnp.int32),
            pltpu.VMEM((_CH,), jnp.int32),
            pltpu.VMEM((_CH, _D), jnp.float32),
            pltpu.VMEM((_CH,), jnp.int32),
            pltpu.VMEM((_CH,), jnp.int32),
            pltpu.VMEM((_CH, _D), jnp.float32),
            pltpu.VMEM((_DR, _D), jnp.float32),
            pltpu.VMEM((_DR,), jnp.int32),
            pltpu.VMEM_SHARED((_NA, _D), jnp.float32),
            pltpu.SemaphoreType.DMA,
            pltpu.SemaphoreType.DMA,
            pltpu.SemaphoreType.DMA,
            pltpu.SemaphoreType.DMA,
            pltpu.SemaphoreType.DMA,
            pltpu.SemaphoreType.DMA,
        ],
    )
    def k(x_hbm, rows_hbm, cols_hbm, z_hbm, out_hbm,
          rv0, cv0, g0, rv1, cv1, g1, deg_v, didx,
          acc_sh, sg0, sg1, ss0, ss1, si0, si1):
        cid = lax.axis_index("c")
        sid = lax.axis_index("s")
        wid = cid * _NS + sid
        pltpu.sync_copy(z_hbm, acc_sh.at[pl.ds(sid * _RPT, _RPT)])
        pltpu.sync_copy(z_hbm.at[pl.ds(0, _DR)], deg_v)
        iota16 = lax.iota(jnp.int32, 16)
        for m in range(_DR // 16):
            didx[pl.ds(m * 16, 16)] = _DBASE + m * 16 + iota16
        plsc.subcore_barrier()

        cb = wid * _CPT
        masks = [iota16 == kk for kk in range(16)]
        ones16 = jnp.full((16,), 1.0, jnp.float32)

        def count_degrees(idx_ref):
            for kk in range(_CH // 16):
                r16 = idx_ref[pl.ds(kk * 16, 16)]
                hgh = lax.shift_right_logical(r16, 7)
                hgl = lax.bitwise_and(r16, 127)
                for mm in masks:
                    plsc.addupdate_scatter(deg_v, [hgh, hgl], ones16,
                                           mask=mm)

        def idx_start(j, rv, cv, si):
            off = pl.multiple_of((cb + j) * _CH, 8)
            pltpu.make_async_copy(rows_hbm.at[pl.ds(off, _CH)], rv, si).start()
            pltpu.make_async_copy(cols_hbm.at[pl.ds(off, _CH)], cv, si).start()

        def idx_wait(j, rv, cv, si):
            off = pl.multiple_of((cb + j) * _CH, 8)
            pltpu.make_async_copy(rows_hbm.at[pl.ds(off, _CH)], rv, si).wait()
            pltpu.make_async_copy(cols_hbm.at[pl.ds(off, _CH)], cv, si).wait()

        def gather_start(cv, g, sg):
            pltpu.make_async_copy(x_hbm.at[cv], g, sg).start()

        def gather_wait(cv, g, sg):
            pltpu.make_async_copy(x_hbm.at[cv], g, sg).wait()

        def scat_start(g, rv, ss):
            pltpu.make_async_copy(g, acc_sh.at[rv], ss).start(add=True)

        def scat_wait(g, rv, ss):
            pltpu.make_async_copy(g, acc_sh.at[rv], ss).wait()

        idx_start(0, rv0, cv0, si0)
        idx_wait(0, rv0, cv0, si0)
        gather_start(cv0, g0, sg0)
        idx_start(1, rv1, cv1, si1)
        count_degrees(rv0)

        def step(t, carry):
            j0 = 2 * t
            idx_wait(j0 + 1, rv1, cv1, si1)
            gather_start(cv1, g1, sg1)
            count_degrees(rv1)
            gather_wait(cv0, g0, sg0)
            scat_start(g0, rv0, ss0)
            scat_wait(g0, rv0, ss0)
            idx_start(j0 + 2, rv0, cv0, si0)
            idx_wait(j0 + 2, rv0, cv0, si0)
            gather_start(cv0, g0, sg0)
            count_degrees(rv0)
            gather_wait(cv1, g1, sg1)
            scat_start(g1, rv1, ss1)
            scat_wait(g1, rv1, ss1)
            idx_start(j0 + 3, rv1, cv1, si1)
            return carry

        lax.fori_loop(0, _CPT // 2 - 1, step, 0)
        idx_wait(_CPT - 1, rv1, cv1, si1)
        gather_start(cv1, g1, sg1)
        count_degrees(rv1)
        gather_wait(cv0, g0, sg0)
        scat_start(g0, rv0, ss0)
        scat_wait(g0, rv0, ss0)
        gather_wait(cv1, g1, sg1)
        scat_start(g1, rv1, ss1)
        scat_wait(g1, rv1, ss1)

        @pl.when(wid < _XC)
        def _():
            off = pl.multiple_of((_CPT * _NW + wid) * _CH, 8)
            pltpu.sync_copy(rows_hbm.at[pl.ds(off, _CH)], rv0)
            pltpu.sync_copy(cols_hbm.at[pl.ds(off, _CH)], cv0)
            gather_start(cv0, g0, sg0)
            count_degrees(rv0)
            gather_wait(cv0, g0, sg0)
            scat_start(g0, rv0, ss0)
            scat_wait(g0, rv0, ss0)

        pltpu.sync_copy(deg_v, acc_sh.at[didx], add=True)
        plsc.subcore_barrier()
        pltpu.sync_copy(acc_sh.at[pl.ds(sid * _RPT, _RPT)],
                        out_hbm.at[cid, pl.ds(sid * _RPT, _RPT)])

    return k(x, rows, cols, zrows)


_BLK = 1024


def _tc_dense(acc, deg, x, W, b2):
    def body(acc_ref, deg_ref, x_ref, w_ref, b_ref, o_ref):
        d = jnp.maximum(deg_ref[...], 1.0)
        neigh = (acc_ref[0] + acc_ref[1]) / d
        h = (jnp.dot(x_ref[...], w_ref[:_D, :],
                     preferred_element_type=jnp.float32,
                     precision=lax.Precision.HIGHEST)
             + jnp.dot(neigh, w_ref[_D:, :],
                       preferred_element_type=jnp.float32,
                       precision=lax.Precision.HIGHEST)
             + b_ref[...])
        o_ref[...] = jnp.maximum(h, 0.0)

    return pl.pallas_call(
        body,
        grid=(_N // _BLK + 1,),
        in_specs=[
            pl.BlockSpec((_NC, _BLK, _D), lambda i: (0, i, 0)),
            pl.BlockSpec((_BLK, 1), lambda i: (i, 0)),
            pl.BlockSpec((_BLK, _D), lambda i: (i, 0)),
            pl.BlockSpec((2 * _D, _H), lambda i: (0, 0)),
            pl.BlockSpec((1, _H), lambda i: (0, 0)),
        ],
        out_specs=pl.BlockSpec((_BLK, _H), lambda i: (i, 0)),
        out_shape=jax.ShapeDtypeStruct((_N, _H), jnp.float32),
    )(acc, deg, x, W, b2)


def kernel(x, rows, cols, W, b):
    zrows = jnp.zeros((_RPT, _D), jnp.float32)
    acc = _sc_scatter(x, rows, cols, zrows)
    dd = acc[0, _DBASE:_DBASE + _DR, :] + acc[1, _DBASE:_DBASE + _DR, :]
    deg = dd.reshape(_NP, 1)
    return _tc_dense(acc, deg, x, W, b.reshape(1, _H))


# speedup vs baseline: 3.7881x; 1.1110x over previous
import functools

import jax
import jax.numpy as jnp
from jax import lax
from jax.experimental import pallas as pl
from jax.experimental.pallas import tpu as pltpu
from jax.experimental.pallas import tpu_sc as plsc

_N = 10000
_E = 320000
_D = 128
_H = 128
_NC = 2
_NS = 16
_NW = _NC * _NS
_CH = 128
_CPT = 80
_NCHUNK = _CPT * _NW
_EPAD = _NCHUNK * _CH
_FILL = 10080
_DR = 80
_DBASE = _N
_NA = 10240
_RPT = _NA // _NS
_NP = _DR * _D


def _sc_scatter(x, rows, cols, zrows):
    mesh = plsc.VectorSubcoreMesh(core_axis_name="c", subcore_axis_name="s")

    @functools.partial(
        pl.kernel,
        out_type=pltpu.HBM((_NC, _NA, _D), jnp.float32),
        mesh=mesh,
        compiler_params=pltpu.CompilerParams(needs_layout_passes=False),
        scratch_types=[
            pltpu.VMEM((_CH,), jnp.int32),
            pltpu.VMEM((_CH,), jnp.int32),
            pltpu.VMEM((_CH,), jnp.int32),
            pltpu.VMEM((_CH,), jnp.int32),
            pltpu.VMEM((_CH,), jnp.int32),
            pltpu.VMEM((_CH,), jnp.int32),
            pltpu.VMEM((_CH,), jnp.int32),
            pltpu.VMEM((_CH,), jnp.int32),
            pltpu.VMEM((_CH, _D), jnp.float32),
            pltpu.VMEM((_CH, _D), jnp.float32),
            pltpu.VMEM((_DR, _D), jnp.float32),
            pltpu.VMEM((_DR,), jnp.int32),
            pltpu.VMEM_SHARED((_NA, _D), jnp.float32),
            pltpu.SemaphoreType.DMA,
            pltpu.SemaphoreType.DMA,
            pltpu.SemaphoreType.DMA,
            pltpu.SemaphoreType.DMA,
            pltpu.SemaphoreType.DMA,
            pltpu.SemaphoreType.DMA,
            pltpu.SemaphoreType.DMA,
            pltpu.SemaphoreType.DMA,
        ],
    )
    def k(x_hbm, rows_hbm, cols_hbm, z_hbm, out_hbm,
          rv0, cv0, rv1, cv1, rv2, cv2, rv3, cv3, g0, g1, deg_v, didx,
          acc_sh, sg0, sg1, ss0, ss1, si0, si1, si2, si3):
        cid = lax.axis_index("c")
        sid = lax.axis_index("s")
        wid = cid * _NS + sid
        pltpu.sync_copy(z_hbm, acc_sh.at[pl.ds(sid * _RPT, _RPT)])
        pltpu.sync_copy(z_hbm.at[pl.ds(0, _DR)], deg_v)
        iota16 = lax.iota(jnp.int32, 16)
        for m in range(_DR // 16):
            didx[pl.ds(m * 16, 16)] = _DBASE + m * 16 + iota16
        plsc.subcore_barrier()

        cb = wid * _CPT
        masks = [iota16 == kk for kk in range(16)]
        ones16 = jnp.full((16,), 1.0, jnp.float32)

        def count_degrees(idx_ref):
            for kk in range(_CH // 16):
                r16 = idx_ref[pl.ds(kk * 16, 16)]
                hgh = lax.shift_right_logical(r16, 7)
                hgl = lax.bitwise_and(r16, 127)
                for mm in masks:
                    plsc.addupdate_scatter(deg_v, [hgh, hgl], ones16,
                                           mask=mm)

        def idx_start(j, rv, cv, si):
            off = pl.multiple_of((cb + j) * _CH, 8)
            pltpu.make_async_copy(rows_hbm.at[pl.ds(off, _CH)], rv, si).start()
            pltpu.make_async_copy(cols_hbm.at[pl.ds(off, _CH)], cv, si).start()

        def idx_wait(j, rv, cv, si):
            off = pl.multiple_of((cb + j) * _CH, 8)
            pltpu.make_async_copy(rows_hbm.at[pl.ds(off, _CH)], rv, si).wait()
            pltpu.make_async_copy(cols_hbm.at[pl.ds(off, _CH)], cv, si).wait()

        def gather_start(cv, g, sg):
            pltpu.make_async_copy(x_hbm.at[cv], g, sg).start()

        def gather_wait(cv, g, sg):
            pltpu.make_async_copy(x_hbm.at[cv], g, sg).wait()

        def scat_start(g, rv, ss):
            pltpu.make_async_copy(g, acc_sh.at[rv], ss).start(add=True)

        def scat_wait(g, rv, ss):
            pltpu.make_async_copy(g, acc_sh.at[rv], ss).wait()

        ib = [(rv0, cv0, si0), (rv1, cv1, si1), (rv2, cv2, si2),
              (rv3, cv3, si3)]
        gb = [(g0, sg0, ss0), (g1, sg1, ss1)]

        def chunk_unit(ph, b_sc, j_pre, j_gat, last=False):
            rv_s, cv_s, _ = ib[ph]
            g, sg, ss = gb[b_sc]
            gather_wait(cv_s, g, sg)
            scat_start(g, rv_s, ss)
            scat_wait(g, rv_s, ss)
            if last:
                return
            rv_p, cv_p, si_p = ib[ph]
            idx_start(j_pre, rv_p, cv_p, si_p)
            rv_g, cv_g, si_g = ib[(ph + 2) % 4]
            idx_wait(j_gat, rv_g, cv_g, si_g)
            gather_start(cv_g, g, sg)
            count_degrees(rv_g)

        for j in range(4):
            rv_p, cv_p, si_p = ib[j]
            idx_start(j, rv_p, cv_p, si_p)
        for j in range(2):
            rv_g, cv_g, si_g = ib[j]
            g, sg, _ = gb[j]
            idx_wait(j, rv_g, cv_g, si_g)
            gather_start(cv_g, g, sg)
            count_degrees(rv_g)

        def step(t, carry):
            j0 = 4 * t
            chunk_unit(0, 0, j0 + 4, j0 + 2)
            chunk_unit(1, 1, j0 + 5, j0 + 3)
            chunk_unit(2, 0, j0 + 6, j0 + 4)
            chunk_unit(3, 1, j0 + 7, j0 + 5)
            return carry

        lax.fori_loop(0, _CPT // 4 - 1, step, 0)
        jt = _CPT - 4

        def tail_unit(ph, b_sc, j_gat):
            rv_s, cv_s, _ = ib[ph]
            g, sg, ss = gb[b_sc]
            gather_wait(cv_s, g, sg)
            scat_start(g, rv_s, ss)
            scat_wait(g, rv_s, ss)
            rv_g, cv_g, si_g = ib[(ph + 2) % 4]
            idx_wait(j_gat, rv_g, cv_g, si_g)
            gather_start(cv_g, g, sg)
            count_degrees(rv_g)

        tail_unit(0, 0, jt + 2)
        tail_unit(1, 1, jt + 3)
        chunk_unit(2, 0, 0, 0, last=True)
        chunk_unit(3, 1, 0, 0, last=True)

        pltpu.sync_copy(deg_v, acc_sh.at[didx], add=True)
        plsc.subcore_barrier()
        pltpu.sync_copy(acc_sh.at[pl.ds(sid * _RPT, _RPT)],
                        out_hbm.at[cid, pl.ds(sid * _RPT, _RPT)])

    return k(x, rows, cols, zrows)


_BLK = 1024


def _tc_dense(acc, deg, x, W, b2):
    def body(acc_ref, deg_ref, x_ref, w_ref, b_ref, o_ref):
        d = jnp.maximum(deg_ref[...], 1.0)
        neigh = (acc_ref[0] + acc_ref[1]) / d
        h = (jnp.dot(x_ref[...], w_ref[:_D, :],
                     preferred_element_type=jnp.float32,
                     precision=lax.Precision.HIGHEST)
             + jnp.dot(neigh, w_ref[_D:, :],
                       preferred_element_type=jnp.float32,
                       precision=lax.Precision.HIGHEST)
             + b_ref[...])
        o_ref[...] = jnp.maximum(h, 0.0)

    return pl.pallas_call(
        body,
        grid=(_N // _BLK + 1,),
        in_specs=[
            pl.BlockSpec((_NC, _BLK, _D), lambda i: (0, i, 0)),
            pl.BlockSpec((_BLK, 1), lambda i: (i, 0)),
            pl.BlockSpec((_BLK, _D), lambda i: (i, 0)),
            pl.BlockSpec((2 * _D, _H), lambda i: (0, 0)),
            pl.BlockSpec((1, _H), lambda i: (0, 0)),
        ],
        out_specs=pl.BlockSpec((_BLK, _H), lambda i: (i, 0)),
        out_shape=jax.ShapeDtypeStruct((_N, _H), jnp.float32),
    )(acc, deg, x, W, b2)


def kernel(x, rows, cols, W, b):
    fill = _EPAD - _E
    frows = _FILL + jnp.arange(fill, dtype=jnp.int32) % 160
    rows_p = jnp.concatenate([rows, frows])
    cols_p = jnp.concatenate([cols, jnp.arange(fill, dtype=jnp.int32) % _N])
    zrows = jnp.zeros((_RPT, _D), jnp.float32)
    acc = _sc_scatter(x, rows_p, cols_p, zrows)
    dd = acc[0, _DBASE:_DBASE + _DR, :] + acc[1, _DBASE:_DBASE + _DR, :]
    deg = dd.reshape(_NP, 1)
    return _tc_dense(acc, deg, x, W, b.reshape(1, _H))
